# baseline (speedup 1.0000x reference)
import functools
import math

import jax
import jax.numpy as jnp
from jax import lax
from jax.experimental import pallas as pl
from jax.experimental.pallas import tpu as pltpu
from jax.experimental.pallas import tpu_sc as plsc

D_MODEL = 64
SCALE = math.sqrt(D_MODEL)
LANES = 16
NC, NS = 2, 16
NW = NC * NS
TSPLIT = 128
RPG = 2


def _make_kernel(bsz: int, seq: int):
    assert bsz % NW == 0
    xrows_w = bsz // NW
    assert xrows_w % (2 * RPG) == 0
    ngroups = xrows_w // RPG
    t2 = seq - TSPLIT
    assert 0 < t2 <= TSPLIT
    mesh = plsc.VectorSubcoreMesh(core_axis_name="c", subcore_axis_name="s")

    @functools.partial(
        pl.kernel,
        out_type=jax.ShapeDtypeStruct((bsz, seq, D_MODEL), jnp.float32),
        mesh=mesh,
        scratch_types=[
            pltpu.VMEM((xrows_w, seq), jnp.int32),
            pltpu.VMEM((2, RPG, seq, D_MODEL), jnp.float32),
            pltpu.SemaphoreType.DMA,
            pltpu.SemaphoreType.DMA,
        ],
        compiler_params=pltpu.CompilerParams(use_tc_tiling_on_sc=False),
    )
    def emb_kernel(x_hbm, table_hbm, out_hbm, idx_v, rows_v, sem0, sem1):
        wid = lax.axis_index("s") * NC + lax.axis_index("c")
        base = wid * xrows_w
        sems = (sem0, sem1)

        pltpu.sync_copy(x_hbm.at[pl.ds(base, xrows_w)], idx_v)

        def fire(group, slot, sem):
            for j in range(RPG):
                r = group * RPG + j
                pltpu.async_copy(
                    table_hbm.at[idx_v.at[r, pl.ds(0, TSPLIT)]],
                    rows_v.at[slot, j, pl.ds(0, TSPLIT)],
                    sem,
                )
                pltpu.async_copy(
                    table_hbm.at[idx_v.at[r, pl.ds(TSPLIT, t2)]],
                    rows_v.at[slot, j, pl.ds(TSPLIT, t2)],
                    sem,
                )

        def drain(group, slot, sem):
            for j in range(RPG):
                r = group * RPG + j
                pltpu.make_async_copy(
                    table_hbm.at[idx_v.at[r, pl.ds(0, TSPLIT)]],
                    rows_v.at[slot, j, pl.ds(0, TSPLIT)],
                    sem,
                ).wait()
                pltpu.make_async_copy(
                    table_hbm.at[idx_v.at[r, pl.ds(TSPLIT, t2)]],
                    rows_v.at[slot, j, pl.ds(TSPLIT, t2)],
                    sem,
                ).wait()

        fire(0, 0, sem0)

        @pl.loop(0, ngroups, step=2)
        def group_loop(g):
            for b in range(2):
                cur = g + b
                nxt = 1 - b

                @pl.when(cur + 1 < ngroups)
                def _start_next():
                    fire(cur + 1, nxt, sems[nxt])

                drain(cur, b, sems[b])

                for j in range(RPG):
                    @plsc.parallel_loop(0, seq, unroll=4)
                    def _scale(r):
                        for k in range(D_MODEL // LANES):
                            sl = pl.ds(k * LANES, LANES)
                            rows_v[b, j, r, sl] = rows_v[b, j, r, sl] * SCALE

                pltpu.sync_copy(
                    rows_v.at[b],
                    out_hbm.at[pl.ds(base + cur * RPG, RPG)],
                )

    return emb_kernel


def kernel(x, table):
    b, l = x.shape
    return _make_kernel(b, l)(x.astype(jnp.int32), table)
